# baseline (speedup 1.0000x reference)
import jax
import jax.numpy as jnp
from jax import lax
from jax.experimental import pallas as pl

_M = 256
_D = 128
_BB = 8


def _body(state_ref, z_ref, mem_ref, out_ref, ctr_ref):
    s = state_ref[...]
    ctr_ref[...] = s + 1
    row = lax.rem(s, _M)
    iota = lax.broadcasted_iota(jnp.int32, (_BB, _M, _D), 1)
    mask = iota == row[:, :, None]
    out_ref[...] = jnp.where(mask, z_ref[...][:, None, :], mem_ref[...])


def kernel(z, mem_state, state):
    b, m, d = mem_state.shape
    state2d = state.reshape(b, 1)
    grid = (b // _BB,)
    out_mem, out_ctr = pl.pallas_call(
        _body,
        grid=grid,
        in_specs=[
            pl.BlockSpec((_BB, 1), lambda i: (i, 0)),
            pl.BlockSpec((_BB, d), lambda i: (i, 0)),
            pl.BlockSpec((_BB, m, d), lambda i: (i, 0, 0)),
        ],
        out_specs=[
            pl.BlockSpec((_BB, m, d), lambda i: (i, 0, 0)),
            pl.BlockSpec((_BB, 1), lambda i: (i, 0)),
        ],
        out_shape=[
            jax.ShapeDtypeStruct((b, m, d), mem_state.dtype),
            jax.ShapeDtypeStruct((b, 1), state.dtype),
        ],
    )(state2d, z, mem_state)
    return out_mem, out_ctr.reshape(b)
